# Initial kernel scaffold; baseline (speedup 1.0000x reference)
import jax
import jax.numpy as jnp
from jax.experimental import pallas as pl


def kernel(x, t, proc_part_pcs, instance_label, edge_index, W_gfp, tW, tb, xW, xb, m1W1, m1b1, m1W2, m1b2, m2W1, m2b1, m2W2, m2b2, m3W1, m3b1, m3W2, m3b2):
    raise NotImplementedError("write your pallas kernel here")


# speedup vs baseline: 1.1490x; 1.1490x over previous
import functools

import jax
import jax.numpy as jnp
import numpy as np
from jax import lax
from jax.experimental import pallas as pl
from jax.experimental.pallas import tpu as pltpu

N = 10000
E = 320000
FEAT = 128
IN_DIM = 7
INST = 20
SIGMA = 25.0
F_CAT = FEAT * 3 + INST

NP_ = 10240
NB = 1024
EB = 2560
DOUT3 = 8

_LN_SIG = float(np.log(SIGMA))


def _t0_body(x_ref, t_ref, wg_ref, tW_ref, tb_ref, xW_ref, xb_ref,
             te_ref, h0_ref, sinv_ref):
    tt = t_ref[:, :]
    proj = tt * wg_ref[:, :] * (2.0 * np.pi)
    gf = jnp.concatenate([jnp.sin(proj), jnp.cos(proj)], axis=-1)
    te = jnp.dot(gf, tW_ref[:, :], preferred_element_type=jnp.float32)
    te = te + tb_ref[:, :]
    te = te * (1.0 / (1.0 + jnp.exp(-te)))
    te_ref[:, :] = te
    h0_ref[:, :] = jnp.dot(x_ref[:, :], xW_ref[:, :],
                           preferred_element_type=jnp.float32) + xb_ref[:, :]
    std = jnp.sqrt((jnp.exp(2.0 * _LN_SIG * tt) - 1.0) / (2.0 * _LN_SIG))
    sinv_ref[:, :] = 1.0 / (std + 1e-07)


def _t0(x, t, W_gfp, tW, tb, xW, xb):
    grid = (NP_ // NB,)
    return pl.pallas_call(
        _t0_body,
        grid=grid,
        in_specs=[
            pl.BlockSpec((NB, IN_DIM), lambda i: (i, 0)),
            pl.BlockSpec((NB, 1), lambda i: (i, 0)),
            pl.BlockSpec((1, FEAT // 2), lambda i: (0, 0)),
            pl.BlockSpec((FEAT, FEAT), lambda i: (0, 0)),
            pl.BlockSpec((1, FEAT), lambda i: (0, 0)),
            pl.BlockSpec((IN_DIM, FEAT), lambda i: (0, 0)),
            pl.BlockSpec((1, FEAT), lambda i: (0, 0)),
        ],
        out_specs=[
            pl.BlockSpec((NB, FEAT), lambda i: (i, 0)),
            pl.BlockSpec((NB, FEAT), lambda i: (i, 0)),
            pl.BlockSpec((NB, 1), lambda i: (i, 0)),
        ],
        out_shape=[
            jax.ShapeDtypeStruct((NP_, FEAT), jnp.float32),
            jax.ShapeDtypeStruct((NP_, FEAT), jnp.float32),
            jax.ShapeDtypeStruct((NP_, 1), jnp.float32),
        ],
    )(x, t, W_gfp, tW, tb, xW, xb)


def _ab_body_h(h_ref, te_ref, ppc_ref, inst_ref, w1d_ref, w1b_ref,
               a_ref, b_ref):
    h = h_ref[:, :]
    hc = jnp.concatenate([h, te_ref[:, :], ppc_ref[:, :], inst_ref[:, :]],
                         axis=-1)
    a_ref[:, :] = jnp.dot(hc, w1d_ref[:, :], preferred_element_type=jnp.float32)
    b_ref[:, :] = jnp.dot(hc, w1b_ref[:, :], preferred_element_type=jnp.float32)


def _ab_body_p(p_ref, te_ref, ppc_ref, inst_ref, w1d_ref, w1b_ref,
               a_ref, b_ref):
    h = jnp.maximum(jnp.maximum(p_ref[0, :, :], p_ref[1, :, :]), 0.0)
    hc = jnp.concatenate([h, te_ref[:, :], ppc_ref[:, :], inst_ref[:, :]],
                         axis=-1)
    a_ref[:, :] = jnp.dot(hc, w1d_ref[:, :], preferred_element_type=jnp.float32)
    b_ref[:, :] = jnp.dot(hc, w1b_ref[:, :], preferred_element_type=jnp.float32)


def _ab(h_or_p, te, ppc, inst, w1d, w1b, from_partials):
    grid = (NP_ // NB,)
    if from_partials:
        body = _ab_body_p
        spec0 = pl.BlockSpec((2, NB, FEAT), lambda i: (0, i, 0))
    else:
        body = _ab_body_h
        spec0 = pl.BlockSpec((NB, FEAT), lambda i: (i, 0))
    return pl.pallas_call(
        body,
        grid=grid,
        in_specs=[
            spec0,
            pl.BlockSpec((NB, FEAT), lambda i: (i, 0)),
            pl.BlockSpec((NB, FEAT), lambda i: (i, 0)),
            pl.BlockSpec((NB, INST), lambda i: (i, 0)),
            pl.BlockSpec((F_CAT, FEAT), lambda i: (0, 0)),
            pl.BlockSpec((F_CAT, FEAT), lambda i: (0, 0)),
        ],
        out_specs=[
            pl.BlockSpec((NB, FEAT), lambda i: (i, 0)),
            pl.BlockSpec((NB, FEAT), lambda i: (i, 0)),
        ],
        out_shape=[
            jax.ShapeDtypeStruct((NP_, FEAT), jnp.float32),
            jax.ShapeDtypeStruct((NP_, FEAT), jnp.float32),
        ],
    )(h_or_p, te, ppc, inst, w1d, w1b)


def _mm_body(pre_ref, b1_ref, w2_ref, b2_ref, zt_ref):
    m = jnp.maximum(pre_ref[:, :] + b1_ref[:, :], 0.0)
    z = jnp.dot(m, w2_ref[:, :], preferred_element_type=jnp.float32)
    z = z + b2_ref[:, :]
    zt_ref[:, :] = z.T


def _mm(pre, b1, w2, b2, dout):
    grid = (E // EB,)
    return pl.pallas_call(
        _mm_body,
        grid=grid,
        in_specs=[
            pl.BlockSpec((EB, FEAT), lambda i: (i, 0)),
            pl.BlockSpec((1, FEAT), lambda i: (0, 0)),
            pl.BlockSpec((FEAT, dout), lambda i: (0, 0)),
            pl.BlockSpec((1, dout), lambda i: (0, 0)),
        ],
        out_specs=pl.BlockSpec((dout, EB), lambda i: (0, i)),
        out_shape=jax.ShapeDtypeStruct((dout, E), jnp.float32),
    )(pre, b1, w2, b2)


def _fin_body(p_ref, sinv_ref, o_ref):
    m = jnp.max(p_ref[:, :, :], axis=0)
    m = jnp.where(jnp.isfinite(m), m, 0.0)
    o_ref[:, :] = m * sinv_ref[:, :]


def _fin(p3, sinv, gparts):
    grid = (NP_ // NB,)
    return pl.pallas_call(
        _fin_body,
        grid=grid,
        in_specs=[
            pl.BlockSpec((gparts, NB, DOUT3), lambda i: (0, i, 0)),
            pl.BlockSpec((NB, 1), lambda i: (i, 0)),
        ],
        out_specs=pl.BlockSpec((NB, DOUT3), lambda i: (i, 0)),
        out_shape=jax.ShapeDtypeStruct((NP_, DOUT3), jnp.float32),
    )(p3, sinv)


def _gather_pre(a, b, src, dst):
    return jnp.take(a, dst, axis=0) + jnp.take(b, src, axis=0)


def _segmax(zt, dst, gparts, dout):
    eg = E // gparts
    parts = []
    for g in range(gparts):
        z = zt[:, g * eg:(g + 1) * eg].T
        d = dst[g * eg:(g + 1) * eg]
        parts.append(jax.ops.segment_max(z, d, num_segments=NP_))
    return jnp.stack(parts, axis=0)


def kernel(x, t, proc_part_pcs, instance_label, edge_index, W_gfp, tW, tb,
           xW, xb, m1W1, m1b1, m1W2, m1b2, m2W1, m2b1, m2W2, m2b2,
           m3W1, m3b1, m3W2, m3b2):
    f32 = jnp.float32
    pad_n = NP_ - N
    xp = jnp.pad(x, ((0, pad_n), (0, 0)))
    tp = jnp.pad(t, ((0, pad_n), (0, 0)))
    ppc = jnp.pad(proc_part_pcs, ((0, pad_n), (0, 0)))
    inst = jnp.pad(instance_label, ((0, pad_n), (0, 0)))
    src = edge_index[0]
    dst = edge_index[1]

    te, h0, sinv = _t0(xp, tp, W_gfp.reshape(1, -1).astype(f32), tW,
                       tb.reshape(1, -1), xW, xb.reshape(1, -1))

    def split_w1(w1):
        top, bot = w1[:F_CAT], w1[F_CAT:]
        return top - bot, bot

    out_p = None
    layer_ws = [(m1W1, m1b1, m1W2, m1b2), (m2W1, m2b1, m2W2, m2b2),
                (m3W1, m3b1, m3W2, m3b2)]
    h_or_p = h0
    from_partials = False
    for li, (w1, b1, w2, b2) in enumerate(layer_ws):
        w1d, w1b = split_w1(w1)
        a, b = _ab(h_or_p, te, ppc, inst, w1d, w1b, from_partials)
        pre = _gather_pre(a, b, src, dst)
        if li < 2:
            dout, gparts = FEAT, 2
        else:
            dout, gparts = DOUT3, 32
            w2 = jnp.pad(w2, ((0, 0), (0, DOUT3 - w2.shape[1])))
            b2 = jnp.pad(b2, ((0, DOUT3 - b2.shape[0]),))
        zt = _mm(pre, b1.reshape(1, -1), w2, b2.reshape(1, -1), dout)
        parts = _segmax(zt, dst, gparts, dout)
        if li < 2:
            h_or_p = parts
            from_partials = True
        else:
            out_p = parts

    out = _fin(out_p, sinv, 32)
    return out[:N, :7]


# speedup vs baseline: 3.0964x; 2.6950x over previous
import functools

import jax
import jax.numpy as jnp
import numpy as np
from jax import lax
from jax.experimental import pallas as pl
from jax.experimental.pallas import tpu as pltpu
from jax.experimental.pallas import tpu_sc as plsc

N = 10000
E = 320000
FEAT = 128
IN_DIM = 7
INST = 20
SIGMA = 25.0
F_CAT = FEAT * 3 + INST

NP_ = 10240
NB = 1024
EB = 2560
DOUT3 = 8

_LN_SIG = float(np.log(SIGMA))


def _t0_body(x_ref, t_ref, wg_ref, tW_ref, tb_ref, xW_ref, xb_ref,
             te_ref, h0_ref, sinv_ref):
    tt = t_ref[:, :]
    proj = tt * wg_ref[:, :] * (2.0 * np.pi)
    gf = jnp.concatenate([jnp.sin(proj), jnp.cos(proj)], axis=-1)
    te = jnp.dot(gf, tW_ref[:, :], preferred_element_type=jnp.float32)
    te = te + tb_ref[:, :]
    te = te * (1.0 / (1.0 + jnp.exp(-te)))
    te_ref[:, :] = te
    h0_ref[:, :] = jnp.dot(x_ref[:, :], xW_ref[:, :],
                           preferred_element_type=jnp.float32) + xb_ref[:, :]
    std = jnp.sqrt((jnp.exp(2.0 * _LN_SIG * tt) - 1.0) / (2.0 * _LN_SIG))
    sinv_ref[:, :] = 1.0 / (std + 1e-07)


def _t0(x, t, W_gfp, tW, tb, xW, xb):
    grid = (NP_ // NB,)
    return pl.pallas_call(
        _t0_body,
        grid=grid,
        in_specs=[
            pl.BlockSpec((NB, IN_DIM), lambda i: (i, 0)),
            pl.BlockSpec((NB, 1), lambda i: (i, 0)),
            pl.BlockSpec((1, FEAT // 2), lambda i: (0, 0)),
            pl.BlockSpec((FEAT, FEAT), lambda i: (0, 0)),
            pl.BlockSpec((1, FEAT), lambda i: (0, 0)),
            pl.BlockSpec((IN_DIM, FEAT), lambda i: (0, 0)),
            pl.BlockSpec((1, FEAT), lambda i: (0, 0)),
        ],
        out_specs=[
            pl.BlockSpec((NB, FEAT), lambda i: (i, 0)),
            pl.BlockSpec((NB, FEAT), lambda i: (i, 0)),
            pl.BlockSpec((NB, 1), lambda i: (i, 0)),
        ],
        out_shape=[
            jax.ShapeDtypeStruct((NP_, FEAT), jnp.float32),
            jax.ShapeDtypeStruct((NP_, FEAT), jnp.float32),
            jax.ShapeDtypeStruct((NP_, 1), jnp.float32),
        ],
    )(x, t, W_gfp, tW, tb, xW, xb)


def _ab_body_h(h_ref, te_ref, ppc_ref, inst_ref, w1d_ref, w1b_ref,
               a_ref, b_ref):
    h = h_ref[:, :]
    hc = jnp.concatenate([h, te_ref[:, :], ppc_ref[:, :], inst_ref[:, :]],
                         axis=-1)
    a_ref[:, :] = jnp.dot(hc, w1d_ref[:, :], preferred_element_type=jnp.float32)
    b_ref[:, :] = jnp.dot(hc, w1b_ref[:, :], preferred_element_type=jnp.float32)


def _ab_body_p(p_ref, te_ref, ppc_ref, inst_ref, w1d_ref, w1b_ref,
               a_ref, b_ref):
    ht = jnp.maximum(jnp.maximum(p_ref[0, :, :], p_ref[1, :, :]), 0.0)
    h = ht.T
    hc = jnp.concatenate([h, te_ref[:, :], ppc_ref[:, :], inst_ref[:, :]],
                         axis=-1)
    a_ref[:, :] = jnp.dot(hc, w1d_ref[:, :], preferred_element_type=jnp.float32)
    b_ref[:, :] = jnp.dot(hc, w1b_ref[:, :], preferred_element_type=jnp.float32)


def _ab(h_or_p, te, ppc, inst, w1d, w1b, from_partials):
    grid = (NP_ // NB,)
    if from_partials:
        body = _ab_body_p
        spec0 = pl.BlockSpec((2, FEAT, NB), lambda i: (0, 0, i))
    else:
        body = _ab_body_h
        spec0 = pl.BlockSpec((NB, FEAT), lambda i: (i, 0))
    return pl.pallas_call(
        body,
        grid=grid,
        in_specs=[
            spec0,
            pl.BlockSpec((NB, FEAT), lambda i: (i, 0)),
            pl.BlockSpec((NB, FEAT), lambda i: (i, 0)),
            pl.BlockSpec((NB, INST), lambda i: (i, 0)),
            pl.BlockSpec((F_CAT, FEAT), lambda i: (0, 0)),
            pl.BlockSpec((F_CAT, FEAT), lambda i: (0, 0)),
        ],
        out_specs=[
            pl.BlockSpec((NB, FEAT), lambda i: (i, 0)),
            pl.BlockSpec((NB, FEAT), lambda i: (i, 0)),
        ],
        out_shape=[
            jax.ShapeDtypeStruct((NP_, FEAT), jnp.float32),
            jax.ShapeDtypeStruct((NP_, FEAT), jnp.float32),
        ],
    )(h_or_p, te, ppc, inst, w1d, w1b)


def _mm_body(pre_ref, b1_ref, w2_ref, b2_ref, zt_ref):
    m = jnp.maximum(pre_ref[:, :] + b1_ref[:, :], 0.0)
    z = jnp.dot(m, w2_ref[:, :], preferred_element_type=jnp.float32)
    z = z + b2_ref[:, :]
    zt_ref[:, :] = z.T


def _mm(pre, b1, w2, b2, dout):
    grid = (E // EB,)
    return pl.pallas_call(
        _mm_body,
        grid=grid,
        in_specs=[
            pl.BlockSpec((EB, FEAT), lambda i: (i, 0)),
            pl.BlockSpec((1, FEAT), lambda i: (0, 0)),
            pl.BlockSpec((FEAT, dout), lambda i: (0, 0)),
            pl.BlockSpec((1, dout), lambda i: (0, 0)),
        ],
        out_specs=pl.BlockSpec((dout, EB), lambda i: (0, i)),
        out_shape=jax.ShapeDtypeStruct((dout, E), jnp.float32),
    )(pre, b1, w2, b2)


def _fin_body(p_ref, sinv_ref, o_ref):
    m = jnp.max(p_ref[:, :, :], axis=0).T
    m = jnp.where(jnp.isfinite(m), m, 0.0)
    o_ref[:, :] = m * sinv_ref[:, :]


def _fin(p3, sinv, gparts):
    grid = (NP_ // NB,)
    return pl.pallas_call(
        _fin_body,
        grid=grid,
        in_specs=[
            pl.BlockSpec((gparts, DOUT3, NB), lambda i: (0, 0, i)),
            pl.BlockSpec((NB, 1), lambda i: (i, 0)),
        ],
        out_specs=pl.BlockSpec((NB, DOUT3), lambda i: (i, 0)),
        out_shape=jax.ShapeDtypeStruct((NP_, DOUT3), jnp.float32),
    )(p3, sinv)


NWORK = 32
EW = E // NWORK
GC = 400
GK = GC // 100


def _sc_gather_body(a_hbm, b_hbm, dst2_hbm, src2_hbm, pre_hbm,
                    idxd, idxs, bufa, bufb, sema, semb):
    nc = 2
    wid = lax.axis_index("s") * nc + lax.axis_index("c")
    row_w = wid * (EW // 100)

    def chunk(c, _):
        row0 = row_w + c * GK
        pltpu.sync_copy(dst2_hbm.at[pl.ds(row0, GK), :], idxd)
        pltpu.sync_copy(src2_hbm.at[pl.ds(row0, GK), :], idxs)
        cps = []
        for k in range(GK):
            cps.append(pltpu.async_copy(
                a_hbm.at[idxd.at[k]], bufa.at[pl.ds(k * 100, 100), :], sema))
            cps.append(pltpu.async_copy(
                b_hbm.at[idxs.at[k]], bufb.at[pl.ds(k * 100, 100), :], semb))
        for cp in cps:
            cp.wait()

        def row(r, _):
            for cc in range(FEAT // 16):
                s = pl.ds(cc * 16, 16)
                bufa[r, s] = bufa[r, s] + bufb[r, s]
            return 0

        lax.fori_loop(0, GC, row, 0)
        pltpu.sync_copy(bufa, pre_hbm.at[pl.ds(row0 * 100, GC), :])
        return 0

    lax.fori_loop(0, EW // GC, chunk, 0)


def _gather_pre(a, b, src2, dst2):
    mesh = plsc.VectorSubcoreMesh(core_axis_name="c", subcore_axis_name="s")
    f = pl.kernel(
        _sc_gather_body,
        out_type=jax.ShapeDtypeStruct((E, FEAT), jnp.float32),
        mesh=mesh,
        scratch_types=[
            pltpu.VMEM((GK, 100), jnp.int32),
            pltpu.VMEM((GK, 100), jnp.int32),
            pltpu.VMEM((GC, FEAT), jnp.float32),
            pltpu.VMEM((GC, FEAT), jnp.float32),
            pltpu.SemaphoreType.DMA,
            pltpu.SemaphoreType.DMA,
        ],
    )
    return f(a, b, dst2, src2)


BC = 1280


def _make_segmax_body(nf, ng, zr):
    eg = E // ng

    def body(zt_hbm, dst_hbm, neg_hbm, p_hbm, dstbuf, zbuf, acc, dupscr):
        nc = 2
        wid = lax.axis_index("s") * nc + lax.axis_index("c")
        fs = wid // ng
        g = wid % ng

        def chunk(c, _):
            e0 = g * eg + c * BC
            pltpu.sync_copy(dst_hbm.at[pl.ds(e0, BC)], dstbuf)
            pltpu.sync_copy(zt_hbm.at[pl.ds(fs * 8, 8), pl.ds(e0, BC)], zbuf)

            def grp(j, _):
                s = pl.ds(j * 16, 16)
                d16 = dstbuf[s]
                iota = lax.iota(jnp.int32, 16)
                plsc.store_scatter(dupscr, [d16], iota)
                rb = plsc.load_gather(dupscr, [d16])
                has_dup = jnp.any(rb != iota)

                def fast():
                    for f in range(8):
                        fidx = jnp.full((16,), f, jnp.int32)
                        cur = plsc.load_gather(acc, [fidx, d16])
                        v = jnp.maximum(cur, zbuf[f, s])
                        plsc.store_scatter(acc, [fidx, d16], v)

                def slow():
                    def rbody(pend):
                        newpend = jnp.zeros((16,), jnp.bool_)
                        for f in range(8):
                            fidx = jnp.full((16,), f, jnp.int32)
                            zv = zbuf[f, s]
                            cur = plsc.load_gather(acc, [fidx, d16])
                            v = jnp.maximum(cur, zv)
                            plsc.store_scatter(acc, [fidx, d16], v, mask=pend)
                            cur2 = plsc.load_gather(acc, [fidx, d16])
                            newpend = jnp.logical_or(newpend, cur2 < zv)
                        return newpend

                    lax.while_loop(lambda p: jnp.any(p), rbody,
                                   jnp.ones((16,), jnp.bool_))

                lax.cond(has_dup, slow, fast)
                return 0

            lax.fori_loop(0, BC // 16, grp, 0)
            return 0

        @pl.when(wid < nf * ng)
        def _():
            pltpu.sync_copy(neg_hbm, acc)
            lax.fori_loop(0, eg // BC, chunk, 0)
            pltpu.sync_copy(acc, p_hbm.at[g, pl.ds(fs * 8, 8), :])

    return body


def _segmax(zt, dst, gparts, dout):
    nf = dout // 8
    mesh = plsc.VectorSubcoreMesh(core_axis_name="c", subcore_axis_name="s")
    neg = jnp.full((8, NP_), -jnp.inf, jnp.float32)
    f = pl.kernel(
        _make_segmax_body(nf, gparts, dout),
        out_type=jax.ShapeDtypeStruct((gparts, dout, NP_), jnp.float32),
        mesh=mesh,
        compiler_params=pltpu.CompilerParams(needs_layout_passes=False),
        scratch_types=[
            pltpu.VMEM((BC,), jnp.int32),
            pltpu.VMEM((8, BC), jnp.float32),
            pltpu.VMEM((8, NP_), jnp.float32),
            pltpu.VMEM((NP_,), jnp.int32),
        ],
    )
    return f(zt, dst, neg)


def kernel(x, t, proc_part_pcs, instance_label, edge_index, W_gfp, tW, tb,
           xW, xb, m1W1, m1b1, m1W2, m1b2, m2W1, m2b1, m2W2, m2b2,
           m3W1, m3b1, m3W2, m3b2):
    f32 = jnp.float32
    pad_n = NP_ - N
    xp = jnp.pad(x, ((0, pad_n), (0, 0)))
    tp = jnp.pad(t, ((0, pad_n), (0, 0)))
    ppc = jnp.pad(proc_part_pcs, ((0, pad_n), (0, 0)))
    inst = jnp.pad(instance_label, ((0, pad_n), (0, 0)))
    src = edge_index[0]
    dst = edge_index[1]
    src2 = src.reshape(E // 100, 100)
    dst2 = dst.reshape(E // 100, 100)

    te, h0, sinv = _t0(xp, tp, W_gfp.reshape(1, -1).astype(f32), tW,
                       tb.reshape(1, -1), xW, xb.reshape(1, -1))

    def split_w1(w1):
        top, bot = w1[:F_CAT], w1[F_CAT:]
        return top - bot, bot

    out_p = None
    layer_ws = [(m1W1, m1b1, m1W2, m1b2), (m2W1, m2b1, m2W2, m2b2),
                (m3W1, m3b1, m3W2, m3b2)]
    h_or_p = h0
    from_partials = False
    for li, (w1, b1, w2, b2) in enumerate(layer_ws):
        w1d, w1b = split_w1(w1)
        a, b = _ab(h_or_p, te, ppc, inst, w1d, w1b, from_partials)
        pre = _gather_pre(a, b, src2, dst2)
        if li < 2:
            dout, gparts = FEAT, 2
        else:
            dout, gparts = DOUT3, 25
            w2 = jnp.pad(w2, ((0, 0), (0, DOUT3 - w2.shape[1])))
            b2 = jnp.pad(b2, ((0, DOUT3 - b2.shape[0]),))
        zt = _mm(pre, b1.reshape(1, -1), w2, b2.reshape(1, -1), dout)
        parts = _segmax(zt, dst, gparts, dout)
        if li < 2:
            h_or_p = parts
            from_partials = True
        else:
            out_p = parts

    out = _fin(out_p, sinv, 25)
    return out[:N, :7]


# speedup vs baseline: 3.7867x; 1.2229x over previous
import functools

import jax
import jax.numpy as jnp
import numpy as np
from jax import lax
from jax.experimental import pallas as pl
from jax.experimental.pallas import tpu as pltpu
from jax.experimental.pallas import tpu_sc as plsc

N = 10000
E = 320000
FEAT = 128
IN_DIM = 7
INST = 20
SIGMA = 25.0
F_CAT = FEAT * 3 + INST

NP_ = 10240
NB = 1024
EB = 2560
DOUT3 = 8

_LN_SIG = float(np.log(SIGMA))


def _t0_body(x_ref, t_ref, wg_ref, tW_ref, tb_ref, xW_ref, xb_ref,
             te_ref, h0_ref, sinv_ref):
    tt = t_ref[:, :]
    proj = tt * wg_ref[:, :] * (2.0 * np.pi)
    gf = jnp.concatenate([jnp.sin(proj), jnp.cos(proj)], axis=-1)
    te = jnp.dot(gf, tW_ref[:, :], preferred_element_type=jnp.float32)
    te = te + tb_ref[:, :]
    te = te * (1.0 / (1.0 + jnp.exp(-te)))
    te_ref[:, :] = te
    h0_ref[:, :] = jnp.dot(x_ref[:, :], xW_ref[:, :],
                           preferred_element_type=jnp.float32) + xb_ref[:, :]
    std = jnp.sqrt((jnp.exp(2.0 * _LN_SIG * tt) - 1.0) / (2.0 * _LN_SIG))
    sinv_ref[:, :] = 1.0 / (std + 1e-07)


def _t0(x, t, W_gfp, tW, tb, xW, xb):
    grid = (NP_ // NB,)
    return pl.pallas_call(
        _t0_body,
        grid=grid,
        in_specs=[
            pl.BlockSpec((NB, IN_DIM), lambda i: (i, 0)),
            pl.BlockSpec((NB, 1), lambda i: (i, 0)),
            pl.BlockSpec((1, FEAT // 2), lambda i: (0, 0)),
            pl.BlockSpec((FEAT, FEAT), lambda i: (0, 0)),
            pl.BlockSpec((1, FEAT), lambda i: (0, 0)),
            pl.BlockSpec((IN_DIM, FEAT), lambda i: (0, 0)),
            pl.BlockSpec((1, FEAT), lambda i: (0, 0)),
        ],
        out_specs=[
            pl.BlockSpec((NB, FEAT), lambda i: (i, 0)),
            pl.BlockSpec((NB, FEAT), lambda i: (i, 0)),
            pl.BlockSpec((NB, 1), lambda i: (i, 0)),
        ],
        out_shape=[
            jax.ShapeDtypeStruct((NP_, FEAT), jnp.float32),
            jax.ShapeDtypeStruct((NP_, FEAT), jnp.float32),
            jax.ShapeDtypeStruct((NP_, 1), jnp.float32),
        ],
    )(x, t, W_gfp, tW, tb, xW, xb)


def _ab_body_h(h_ref, te_ref, ppc_ref, inst_ref, w1d_ref, w1b_ref,
               a_ref, b_ref):
    h = h_ref[:, :]
    hc = jnp.concatenate([h, te_ref[:, :], ppc_ref[:, :], inst_ref[:, :]],
                         axis=-1)
    a_ref[:, :] = jnp.dot(hc, w1d_ref[:, :], preferred_element_type=jnp.float32)
    b_ref[:, :] = jnp.dot(hc, w1b_ref[:, :], preferred_element_type=jnp.float32)


def _ab_body_p(p_ref, te_ref, ppc_ref, inst_ref, w1d_ref, w1b_ref,
               a_ref, b_ref):
    ht = jnp.maximum(jnp.maximum(p_ref[0, :, :], p_ref[1, :, :]), 0.0)
    h = ht.T
    hc = jnp.concatenate([h, te_ref[:, :], ppc_ref[:, :], inst_ref[:, :]],
                         axis=-1)
    a_ref[:, :] = jnp.dot(hc, w1d_ref[:, :], preferred_element_type=jnp.float32)
    b_ref[:, :] = jnp.dot(hc, w1b_ref[:, :], preferred_element_type=jnp.float32)


def _ab(h_or_p, te, ppc, inst, w1d, w1b, from_partials):
    grid = (NP_ // NB,)
    if from_partials:
        body = _ab_body_p
        spec0 = pl.BlockSpec((2, FEAT, NB), lambda i: (0, 0, i))
    else:
        body = _ab_body_h
        spec0 = pl.BlockSpec((NB, FEAT), lambda i: (i, 0))
    return pl.pallas_call(
        body,
        grid=grid,
        in_specs=[
            spec0,
            pl.BlockSpec((NB, FEAT), lambda i: (i, 0)),
            pl.BlockSpec((NB, FEAT), lambda i: (i, 0)),
            pl.BlockSpec((NB, INST), lambda i: (i, 0)),
            pl.BlockSpec((F_CAT, FEAT), lambda i: (0, 0)),
            pl.BlockSpec((F_CAT, FEAT), lambda i: (0, 0)),
        ],
        out_specs=[
            pl.BlockSpec((NB, FEAT), lambda i: (i, 0)),
            pl.BlockSpec((NB, FEAT), lambda i: (i, 0)),
        ],
        out_shape=[
            jax.ShapeDtypeStruct((NP_, FEAT), jnp.float32),
            jax.ShapeDtypeStruct((NP_, FEAT), jnp.float32),
        ],
    )(h_or_p, te, ppc, inst, w1d, w1b)


def _mm_body(pre_ref, b1_ref, w2_ref, b2_ref, zt_ref):
    m = jnp.maximum(pre_ref[:, :] + b1_ref[:, :], 0.0)
    z = jnp.dot(m, w2_ref[:, :], preferred_element_type=jnp.float32)
    z = z + b2_ref[:, :]
    zt_ref[:, :] = z.T


def _mm(pre, b1, w2, b2, dout):
    grid = (E // EB,)
    return pl.pallas_call(
        _mm_body,
        grid=grid,
        in_specs=[
            pl.BlockSpec((EB, FEAT), lambda i: (i, 0)),
            pl.BlockSpec((1, FEAT), lambda i: (0, 0)),
            pl.BlockSpec((FEAT, dout), lambda i: (0, 0)),
            pl.BlockSpec((1, dout), lambda i: (0, 0)),
        ],
        out_specs=pl.BlockSpec((dout, EB), lambda i: (0, i)),
        out_shape=jax.ShapeDtypeStruct((dout, E), jnp.float32),
    )(pre, b1, w2, b2)


def _fin_body(p_ref, sinv_ref, o_ref):
    m = jnp.max(p_ref[:, :, :], axis=0).T
    m = jnp.where(jnp.isfinite(m), m, 0.0)
    o_ref[:, :] = m * sinv_ref[:, :]


def _fin(p3, sinv, gparts):
    grid = (NP_ // NB,)
    return pl.pallas_call(
        _fin_body,
        grid=grid,
        in_specs=[
            pl.BlockSpec((gparts, DOUT3, NB), lambda i: (0, 0, i)),
            pl.BlockSpec((NB, 1), lambda i: (i, 0)),
        ],
        out_specs=pl.BlockSpec((NB, DOUT3), lambda i: (i, 0)),
        out_shape=jax.ShapeDtypeStruct((NP_, DOUT3), jnp.float32),
    )(p3, sinv)


NWORK = 32
EW = E // NWORK
GC = 200
GK = GC // 100


def _sc_gather_body(a_hbm, b_hbm, dst2_hbm, src2_hbm, pre_hbm,
                    idxd, idxs, bufa, bufb, sema, semb):
    nc = 2
    wid = lax.axis_index("s") * nc + lax.axis_index("c")
    row_w = wid * (EW // 100)
    nch = EW // GC

    def fire(c, p):
        row0 = row_w + c * GK
        pltpu.sync_copy(dst2_hbm.at[pl.ds(row0, GK), :], idxd.at[p])
        pltpu.sync_copy(src2_hbm.at[pl.ds(row0, GK), :], idxs.at[p])
        for k in range(GK):
            pltpu.async_copy(a_hbm.at[idxd.at[p, k]],
                             bufa.at[p, pl.ds(k * 100, 100), :], sema[p])
            pltpu.async_copy(b_hbm.at[idxs.at[p, k]],
                             bufb.at[p, pl.ds(k * 100, 100), :], semb[p])

    def drain_process(c, p):
        for k in range(GK):
            pltpu.make_async_copy(a_hbm.at[idxd.at[p, k]],
                                  bufa.at[p, pl.ds(k * 100, 100), :],
                                  sema[p]).wait()
            pltpu.make_async_copy(b_hbm.at[idxs.at[p, k]],
                                  bufb.at[p, pl.ds(k * 100, 100), :],
                                  semb[p]).wait()

        def row(r, _):
            for cc in range(FEAT // 16):
                s = pl.ds(cc * 16, 16)
                bufa[p, r, s] = bufa[p, r, s] + bufb[p, r, s]
            return 0

        lax.fori_loop(0, GC, row, 0)
        pltpu.sync_copy(bufa.at[p],
                        pre_hbm.at[pl.ds(wid * EW + c * GC, GC), :])

    fire(0, 0)

    def pair(i, _):
        c0 = 2 * i
        fire(c0 + 1, 1)
        drain_process(c0, 0)

        @pl.when(c0 + 2 < nch)
        def _():
            fire(c0 + 2, 0)

        drain_process(c0 + 1, 1)
        return 0

    lax.fori_loop(0, nch // 2, pair, 0)


def _gather_pre(a, b, src2, dst2):
    mesh = plsc.VectorSubcoreMesh(core_axis_name="c", subcore_axis_name="s")
    f = pl.kernel(
        _sc_gather_body,
        out_type=jax.ShapeDtypeStruct((E, FEAT), jnp.float32),
        mesh=mesh,
        scratch_types=[
            pltpu.VMEM((2, GK, 100), jnp.int32),
            pltpu.VMEM((2, GK, 100), jnp.int32),
            pltpu.VMEM((2, GC, FEAT), jnp.float32),
            pltpu.VMEM((2, GC, FEAT), jnp.float32),
            [pltpu.SemaphoreType.DMA, pltpu.SemaphoreType.DMA],
            [pltpu.SemaphoreType.DMA, pltpu.SemaphoreType.DMA],
        ],
    )
    return f(a, b, dst2, src2)


BC = 1280


def _make_segmax_body(nf, ng, zr):
    eg = E // ng

    def body(zt_hbm, dst_hbm, neg_hbm, p_hbm, dstbuf, zbuf, acc, dupscr,
             semd, semz):
        nc = 2
        wid = lax.axis_index("s") * nc + lax.axis_index("c")
        fs = wid // ng
        g = wid % ng
        nch = eg // BC

        def fire(c, p):
            e0 = g * eg + c * BC
            pltpu.async_copy(dst_hbm.at[pl.ds(e0, BC)], dstbuf.at[p], semd[p])
            pltpu.async_copy(zt_hbm.at[pl.ds(fs * 8, 8), pl.ds(e0, BC)],
                             zbuf.at[p], semz[p])

        def chunk(c, p):
            e0 = g * eg + c * BC
            pltpu.make_async_copy(dst_hbm.at[pl.ds(e0, BC)], dstbuf.at[p],
                                  semd[p]).wait()
            pltpu.make_async_copy(zt_hbm.at[pl.ds(fs * 8, 8), pl.ds(e0, BC)],
                                  zbuf.at[p], semz[p]).wait()

            def grp(j, _):
                s = pl.ds(j * 16, 16)
                d16 = dstbuf[p, s]
                iota = lax.iota(jnp.int32, 16)
                plsc.store_scatter(dupscr, [d16], iota)
                rb = plsc.load_gather(dupscr, [d16])
                has_dup = jnp.any(rb != iota)

                def fast():
                    for f in range(8):
                        fidx = jnp.full((16,), f, jnp.int32)
                        cur = plsc.load_gather(acc, [fidx, d16])
                        v = jnp.maximum(cur, zbuf[p, f, s])
                        plsc.store_scatter(acc, [fidx, d16], v)

                def slow():
                    def rbody(pend):
                        newpend = jnp.zeros((16,), jnp.bool_)
                        for f in range(8):
                            fidx = jnp.full((16,), f, jnp.int32)
                            zv = zbuf[p, f, s]
                            cur = plsc.load_gather(acc, [fidx, d16])
                            v = jnp.maximum(cur, zv)
                            plsc.store_scatter(acc, [fidx, d16], v, mask=pend)
                            cur2 = plsc.load_gather(acc, [fidx, d16])
                            newpend = jnp.logical_or(newpend, cur2 < zv)
                        return newpend

                    lax.while_loop(lambda p: jnp.any(p), rbody,
                                   jnp.ones((16,), jnp.bool_))

                lax.cond(has_dup, slow, fast)
                return 0

            lax.fori_loop(0, BC // 16, grp, 0)

        @pl.when(wid < nf * ng)
        def _():
            pltpu.sync_copy(neg_hbm, acc)
            fire(0, 0)

            def pair(i, _):
                c0 = 2 * i

                @pl.when(c0 + 1 < nch)
                def _():
                    fire(c0 + 1, 1)

                chunk(c0, 0)

                @pl.when(c0 + 2 < nch)
                def _():
                    fire(c0 + 2, 0)

                @pl.when(c0 + 1 < nch)
                def _():
                    chunk(c0 + 1, 1)

                return 0

            lax.fori_loop(0, (nch + 1) // 2, pair, 0)
            pltpu.sync_copy(acc, p_hbm.at[g, pl.ds(fs * 8, 8), :])

    return body


def _segmax(zt, dst, gparts, dout):
    nf = dout // 8
    mesh = plsc.VectorSubcoreMesh(core_axis_name="c", subcore_axis_name="s")
    neg = jnp.full((8, NP_), -jnp.inf, jnp.float32)
    f = pl.kernel(
        _make_segmax_body(nf, gparts, dout),
        out_type=jax.ShapeDtypeStruct((gparts, dout, NP_), jnp.float32),
        mesh=mesh,
        compiler_params=pltpu.CompilerParams(needs_layout_passes=False),
        scratch_types=[
            pltpu.VMEM((2, BC), jnp.int32),
            pltpu.VMEM((2, 8, BC), jnp.float32),
            pltpu.VMEM((8, NP_), jnp.float32),
            pltpu.VMEM((NP_,), jnp.int32),
            [pltpu.SemaphoreType.DMA, pltpu.SemaphoreType.DMA],
            [pltpu.SemaphoreType.DMA, pltpu.SemaphoreType.DMA],
        ],
    )
    return f(zt, dst, neg)


def kernel(x, t, proc_part_pcs, instance_label, edge_index, W_gfp, tW, tb,
           xW, xb, m1W1, m1b1, m1W2, m1b2, m2W1, m2b1, m2W2, m2b2,
           m3W1, m3b1, m3W2, m3b2):
    f32 = jnp.float32
    pad_n = NP_ - N
    xp = jnp.pad(x, ((0, pad_n), (0, 0)))
    tp = jnp.pad(t, ((0, pad_n), (0, 0)))
    ppc = jnp.pad(proc_part_pcs, ((0, pad_n), (0, 0)))
    inst = jnp.pad(instance_label, ((0, pad_n), (0, 0)))
    src = edge_index[0]
    dst = edge_index[1]
    src2 = src.reshape(E // 100, 100)
    dst2 = dst.reshape(E // 100, 100)

    te, h0, sinv = _t0(xp, tp, W_gfp.reshape(1, -1).astype(f32), tW,
                       tb.reshape(1, -1), xW, xb.reshape(1, -1))

    def split_w1(w1):
        top, bot = w1[:F_CAT], w1[F_CAT:]
        return top - bot, bot

    out_p = None
    layer_ws = [(m1W1, m1b1, m1W2, m1b2), (m2W1, m2b1, m2W2, m2b2),
                (m3W1, m3b1, m3W2, m3b2)]
    h_or_p = h0
    from_partials = False
    for li, (w1, b1, w2, b2) in enumerate(layer_ws):
        w1d, w1b = split_w1(w1)
        a, b = _ab(h_or_p, te, ppc, inst, w1d, w1b, from_partials)
        pre = _gather_pre(a, b, src2, dst2)
        if li < 2:
            dout, gparts = FEAT, 2
        else:
            dout, gparts = DOUT3, 25
            w2 = jnp.pad(w2, ((0, 0), (0, DOUT3 - w2.shape[1])))
            b2 = jnp.pad(b2, ((0, DOUT3 - b2.shape[0]),))
        zt = _mm(pre, b1.reshape(1, -1), w2, b2.reshape(1, -1), dout)
        parts = _segmax(zt, dst, gparts, dout)
        if li < 2:
            h_or_p = parts
            from_partials = True
        else:
            out_p = parts

    out = _fin(out_p, sinv, 25)
    return out[:N, :7]


# speedup vs baseline: 5.3030x; 1.4004x over previous
import functools

import jax
import jax.numpy as jnp
import numpy as np
from jax import lax
from jax.experimental import pallas as pl
from jax.experimental.pallas import tpu as pltpu
from jax.experimental.pallas import tpu_sc as plsc

N = 10000
E = 320000
FEAT = 128
IN_DIM = 7
INST = 20
SIGMA = 25.0
F_CAT = FEAT * 3 + INST

NP_ = 10240
NB = 1024
EB = 2560
DOUT3 = 8

_LN_SIG = float(np.log(SIGMA))


def _t0_body(x_ref, t_ref, wg_ref, tW_ref, tb_ref, xW_ref, xb_ref,
             te_ref, h0_ref, sinv_ref):
    tt = t_ref[:, :]
    proj = tt * wg_ref[:, :] * (2.0 * np.pi)
    gf = jnp.concatenate([jnp.sin(proj), jnp.cos(proj)], axis=-1)
    te = jnp.dot(gf, tW_ref[:, :], preferred_element_type=jnp.float32)
    te = te + tb_ref[:, :]
    te = te * (1.0 / (1.0 + jnp.exp(-te)))
    te_ref[:, :] = te
    h0_ref[:, :] = jnp.dot(x_ref[:, :], xW_ref[:, :],
                           preferred_element_type=jnp.float32) + xb_ref[:, :]
    std = jnp.sqrt((jnp.exp(2.0 * _LN_SIG * tt) - 1.0) / (2.0 * _LN_SIG))
    sinv_ref[:, :] = 1.0 / (std + 1e-07)


def _t0(x, t, W_gfp, tW, tb, xW, xb):
    grid = (NP_ // NB,)
    return pl.pallas_call(
        _t0_body,
        grid=grid,
        in_specs=[
            pl.BlockSpec((NB, IN_DIM), lambda i: (i, 0)),
            pl.BlockSpec((NB, 1), lambda i: (i, 0)),
            pl.BlockSpec((1, FEAT // 2), lambda i: (0, 0)),
            pl.BlockSpec((FEAT, FEAT), lambda i: (0, 0)),
            pl.BlockSpec((1, FEAT), lambda i: (0, 0)),
            pl.BlockSpec((IN_DIM, FEAT), lambda i: (0, 0)),
            pl.BlockSpec((1, FEAT), lambda i: (0, 0)),
        ],
        out_specs=[
            pl.BlockSpec((NB, FEAT), lambda i: (i, 0)),
            pl.BlockSpec((NB, FEAT), lambda i: (i, 0)),
            pl.BlockSpec((NB, 1), lambda i: (i, 0)),
        ],
        out_shape=[
            jax.ShapeDtypeStruct((NP_, FEAT), jnp.float32),
            jax.ShapeDtypeStruct((NP_, FEAT), jnp.float32),
            jax.ShapeDtypeStruct((NP_, 1), jnp.float32),
        ],
    )(x, t, W_gfp, tW, tb, xW, xb)


def _ab_body_h(h_ref, te_ref, ppc_ref, inst_ref, w1d_ref, w1b_ref,
               a_ref, b_ref):
    h = h_ref[:, :]
    hc = jnp.concatenate([h, te_ref[:, :], ppc_ref[:, :], inst_ref[:, :]],
                         axis=-1)
    a_ref[:, :] = jnp.dot(hc, w1d_ref[:, :], preferred_element_type=jnp.float32)
    b_ref[:, :] = jnp.dot(hc, w1b_ref[:, :], preferred_element_type=jnp.float32)


def _ab_body_p(p_ref, te_ref, ppc_ref, inst_ref, w1d_ref, w1b_ref,
               a_ref, b_ref):
    ht = jnp.maximum(jnp.maximum(p_ref[0, :, :], p_ref[1, :, :]), 0.0)
    h = ht.T
    hc = jnp.concatenate([h, te_ref[:, :], ppc_ref[:, :], inst_ref[:, :]],
                         axis=-1)
    a_ref[:, :] = jnp.dot(hc, w1d_ref[:, :], preferred_element_type=jnp.float32)
    b_ref[:, :] = jnp.dot(hc, w1b_ref[:, :], preferred_element_type=jnp.float32)


def _ab(h_or_p, te, ppc, inst, w1d, w1b, from_partials):
    grid = (NP_ // NB,)
    if from_partials:
        body = _ab_body_p
        spec0 = pl.BlockSpec((2, FEAT, NB), lambda i: (0, 0, i))
    else:
        body = _ab_body_h
        spec0 = pl.BlockSpec((NB, FEAT), lambda i: (i, 0))
    return pl.pallas_call(
        body,
        grid=grid,
        in_specs=[
            spec0,
            pl.BlockSpec((NB, FEAT), lambda i: (i, 0)),
            pl.BlockSpec((NB, FEAT), lambda i: (i, 0)),
            pl.BlockSpec((NB, INST), lambda i: (i, 0)),
            pl.BlockSpec((F_CAT, FEAT), lambda i: (0, 0)),
            pl.BlockSpec((F_CAT, FEAT), lambda i: (0, 0)),
        ],
        out_specs=[
            pl.BlockSpec((NB, FEAT), lambda i: (i, 0)),
            pl.BlockSpec((NB, FEAT), lambda i: (i, 0)),
        ],
        out_shape=[
            jax.ShapeDtypeStruct((NP_, FEAT), jnp.float32),
            jax.ShapeDtypeStruct((NP_, FEAT), jnp.float32),
        ],
    )(h_or_p, te, ppc, inst, w1d, w1b)


def _mm_body(pre_ref, b1_ref, w2_ref, b2_ref, zt_ref):
    m = jnp.maximum(pre_ref[:, :] + b1_ref[:, :], 0.0)
    z = jnp.dot(m, w2_ref[:, :], preferred_element_type=jnp.float32)
    z = z + b2_ref[:, :]
    zt_ref[:, :] = z.T


def _mm(pre, b1, w2, b2, dout):
    grid = (E // EB,)
    return pl.pallas_call(
        _mm_body,
        grid=grid,
        in_specs=[
            pl.BlockSpec((EB, FEAT), lambda i: (i, 0)),
            pl.BlockSpec((1, FEAT), lambda i: (0, 0)),
            pl.BlockSpec((FEAT, dout), lambda i: (0, 0)),
            pl.BlockSpec((1, dout), lambda i: (0, 0)),
        ],
        out_specs=pl.BlockSpec((dout, EB), lambda i: (0, i)),
        out_shape=jax.ShapeDtypeStruct((dout, E), jnp.float32),
    )(pre, b1, w2, b2)


def _fin_body(p_ref, sinv_ref, o_ref):
    m = jnp.max(p_ref[:, :, :], axis=0).T
    m = jnp.where(jnp.isfinite(m), m, 0.0)
    o_ref[:, :] = m * sinv_ref[:, :]


def _fin(p3, sinv, gparts):
    grid = (NP_ // NB,)
    return pl.pallas_call(
        _fin_body,
        grid=grid,
        in_specs=[
            pl.BlockSpec((gparts, DOUT3, NB), lambda i: (0, 0, i)),
            pl.BlockSpec((NB, 1), lambda i: (i, 0)),
        ],
        out_specs=pl.BlockSpec((NB, DOUT3), lambda i: (i, 0)),
        out_shape=jax.ShapeDtypeStruct((NP_, DOUT3), jnp.float32),
    )(p3, sinv)


NWORK = 32
EW = E // NWORK
GC = 200
GK = GC // 100


def _sc_gather_body(a_hbm, b_hbm, dst2_hbm, src2_hbm, pre_hbm,
                    idxd, idxs, bufa, bufb, sema, semb):
    nc = 2
    wid = lax.axis_index("s") * nc + lax.axis_index("c")
    row_w = wid * (EW // 100)
    nch = EW // GC

    def fire(c, p):
        row0 = row_w + c * GK
        pltpu.sync_copy(dst2_hbm.at[pl.ds(row0, GK), :], idxd.at[p])
        pltpu.sync_copy(src2_hbm.at[pl.ds(row0, GK), :], idxs.at[p])
        for k in range(GK):
            pltpu.async_copy(a_hbm.at[idxd.at[p, k]],
                             bufa.at[p, pl.ds(k * 100, 100), :], sema[p])
            pltpu.async_copy(b_hbm.at[idxs.at[p, k]],
                             bufb.at[p, pl.ds(k * 100, 100), :], semb[p])

    def drain_process(c, p):
        for k in range(GK):
            pltpu.make_async_copy(a_hbm.at[idxd.at[p, k]],
                                  bufa.at[p, pl.ds(k * 100, 100), :],
                                  sema[p]).wait()
            pltpu.make_async_copy(b_hbm.at[idxs.at[p, k]],
                                  bufb.at[p, pl.ds(k * 100, 100), :],
                                  semb[p]).wait()

        def row(r, _):
            for cc in range(FEAT // 16):
                s = pl.ds(cc * 16, 16)
                bufa[p, r, s] = bufa[p, r, s] + bufb[p, r, s]
            return 0

        lax.fori_loop(0, GC, row, 0)
        pltpu.sync_copy(bufa.at[p],
                        pre_hbm.at[pl.ds(wid * EW + c * GC, GC), :])

    fire(0, 0)

    def pair(i, _):
        c0 = 2 * i
        fire(c0 + 1, 1)
        drain_process(c0, 0)

        @pl.when(c0 + 2 < nch)
        def _():
            fire(c0 + 2, 0)

        drain_process(c0 + 1, 1)
        return 0

    lax.fori_loop(0, nch // 2, pair, 0)


def _gather_pre(a, b, src2, dst2):
    mesh = plsc.VectorSubcoreMesh(core_axis_name="c", subcore_axis_name="s")
    f = pl.kernel(
        _sc_gather_body,
        out_type=jax.ShapeDtypeStruct((E, FEAT), jnp.float32),
        mesh=mesh,
        scratch_types=[
            pltpu.VMEM((2, GK, 100), jnp.int32),
            pltpu.VMEM((2, GK, 100), jnp.int32),
            pltpu.VMEM((2, GC, FEAT), jnp.float32),
            pltpu.VMEM((2, GC, FEAT), jnp.float32),
            [pltpu.SemaphoreType.DMA, pltpu.SemaphoreType.DMA],
            [pltpu.SemaphoreType.DMA, pltpu.SemaphoreType.DMA],
        ],
    )
    return f(a, b, dst2, src2)


BC = 1280


def _make_segmax_body(nf, ng, zr):
    eg = E // ng

    def body(zt_hbm, dst_hbm, neg_hbm, p_hbm, dstbuf, zbuf, acc, dupscr,
             semd, semz):
        nc = 2
        wid = lax.axis_index("s") * nc + lax.axis_index("c")
        fs = wid // ng
        g = wid % ng
        nch = eg // BC

        def fire(c, p):
            e0 = g * eg + c * BC
            pltpu.async_copy(dst_hbm.at[pl.ds(e0, BC)], dstbuf.at[p], semd[p])
            pltpu.async_copy(zt_hbm.at[pl.ds(fs * 8, 8), pl.ds(e0, BC)],
                             zbuf.at[p], semz[p])

        def chunk(c, p):
            e0 = g * eg + c * BC
            pltpu.make_async_copy(dst_hbm.at[pl.ds(e0, BC)], dstbuf.at[p],
                                  semd[p]).wait()
            pltpu.make_async_copy(zt_hbm.at[pl.ds(fs * 8, 8), pl.ds(e0, BC)],
                                  zbuf.at[p], semz[p]).wait()

            iota = lax.iota(jnp.int32, 16)
            fidxs = [jnp.full((16,), f, jnp.int32) for f in range(8)]

            def grp4(jb, _):
                pend_any = jnp.zeros((16,), jnp.bool_)
                for u in range(4):
                    j = jb * 4 + u
                    s = pl.ds(j * 16, 16)
                    d16 = dstbuf[p, s]
                    plsc.store_scatter(dupscr, [d16], iota)
                    rb = plsc.load_gather(dupscr, [d16])
                    win = rb == iota
                    curs = [plsc.load_gather(acc, [fidxs[f], d16])
                            for f in range(8)]
                    zvs = [zbuf[p, f, s] for f in range(8)]
                    for f in range(8):
                        plsc.store_scatter(acc, [fidxs[f], d16],
                                           jnp.maximum(curs[f], zvs[f]),
                                           mask=win)
                    pend_any = jnp.logical_or(pend_any,
                                              jnp.logical_not(win))

                def slow():
                    for u in range(4):
                        j = jb * 4 + u
                        s = pl.ds(j * 16, 16)
                        d16 = dstbuf[p, s]

                        def rbody(pend):
                            newpend = jnp.zeros((16,), jnp.bool_)
                            for f in range(8):
                                zv = zbuf[p, f, s]
                                cur = plsc.load_gather(acc, [fidxs[f], d16])
                                v = jnp.maximum(cur, zv)
                                plsc.store_scatter(acc, [fidxs[f], d16], v,
                                                   mask=pend)
                                cur2 = plsc.load_gather(acc, [fidxs[f], d16])
                                newpend = jnp.logical_or(newpend, cur2 < zv)
                            return newpend

                        lax.while_loop(lambda q: jnp.any(q), rbody,
                                       jnp.ones((16,), jnp.bool_))

                lax.cond(jnp.any(pend_any), slow, lambda: None)
                return 0

            lax.fori_loop(0, BC // 64, grp4, 0)

        @pl.when(wid < nf * ng)
        def _():
            pltpu.sync_copy(neg_hbm, acc)
            fire(0, 0)

            def pair(i, _):
                c0 = 2 * i

                @pl.when(c0 + 1 < nch)
                def _():
                    fire(c0 + 1, 1)

                chunk(c0, 0)

                @pl.when(c0 + 2 < nch)
                def _():
                    fire(c0 + 2, 0)

                @pl.when(c0 + 1 < nch)
                def _():
                    chunk(c0 + 1, 1)

                return 0

            lax.fori_loop(0, (nch + 1) // 2, pair, 0)
            pltpu.sync_copy(acc, p_hbm.at[g, pl.ds(fs * 8, 8), :])

    return body


def _segmax(zt, dst, gparts, dout):
    nf = dout // 8
    mesh = plsc.VectorSubcoreMesh(core_axis_name="c", subcore_axis_name="s")
    neg = jnp.full((8, NP_), -jnp.inf, jnp.float32)
    f = pl.kernel(
        _make_segmax_body(nf, gparts, dout),
        out_type=jax.ShapeDtypeStruct((gparts, dout, NP_), jnp.float32),
        mesh=mesh,
        compiler_params=pltpu.CompilerParams(needs_layout_passes=False),
        scratch_types=[
            pltpu.VMEM((2, BC), jnp.int32),
            pltpu.VMEM((2, 8, BC), jnp.float32),
            pltpu.VMEM((8, NP_), jnp.float32),
            pltpu.VMEM((NP_,), jnp.int32),
            [pltpu.SemaphoreType.DMA, pltpu.SemaphoreType.DMA],
            [pltpu.SemaphoreType.DMA, pltpu.SemaphoreType.DMA],
        ],
    )
    return f(zt, dst, neg)


def kernel(x, t, proc_part_pcs, instance_label, edge_index, W_gfp, tW, tb,
           xW, xb, m1W1, m1b1, m1W2, m1b2, m2W1, m2b1, m2W2, m2b2,
           m3W1, m3b1, m3W2, m3b2):
    f32 = jnp.float32
    pad_n = NP_ - N
    xp = jnp.pad(x, ((0, pad_n), (0, 0)))
    tp = jnp.pad(t, ((0, pad_n), (0, 0)))
    ppc = jnp.pad(proc_part_pcs, ((0, pad_n), (0, 0)))
    inst = jnp.pad(instance_label, ((0, pad_n), (0, 0)))
    src = edge_index[0]
    dst = edge_index[1]
    src2 = src.reshape(E // 100, 100)
    dst2 = dst.reshape(E // 100, 100)

    te, h0, sinv = _t0(xp, tp, W_gfp.reshape(1, -1).astype(f32), tW,
                       tb.reshape(1, -1), xW, xb.reshape(1, -1))

    def split_w1(w1):
        top, bot = w1[:F_CAT], w1[F_CAT:]
        return top - bot, bot

    out_p = None
    layer_ws = [(m1W1, m1b1, m1W2, m1b2), (m2W1, m2b1, m2W2, m2b2),
                (m3W1, m3b1, m3W2, m3b2)]
    h_or_p = h0
    from_partials = False
    for li, (w1, b1, w2, b2) in enumerate(layer_ws):
        w1d, w1b = split_w1(w1)
        a, b = _ab(h_or_p, te, ppc, inst, w1d, w1b, from_partials)
        pre = _gather_pre(a, b, src2, dst2)
        if li < 2:
            dout, gparts = FEAT, 2
        else:
            dout, gparts = DOUT3, 25
            w2 = jnp.pad(w2, ((0, 0), (0, DOUT3 - w2.shape[1])))
            b2 = jnp.pad(b2, ((0, DOUT3 - b2.shape[0]),))
        zt = _mm(pre, b1.reshape(1, -1), w2, b2.reshape(1, -1), dout)
        parts = _segmax(zt, dst, gparts, dout)
        if li < 2:
            h_or_p = parts
            from_partials = True
        else:
            out_p = parts

    out = _fin(out_p, sinv, 25)
    return out[:N, :7]
